# baseline (speedup 1.0000x reference)
import jax
import jax.numpy as jnp
from jax.experimental import pallas as pl
from jax.experimental.pallas import tpu as pltpu

_BM = 200
_SLOTS = 4


def _gcn_manual(x_ref, w_ref, adj_hbm, out_ref, support_ref, adj_buf, sem):
    i = pl.program_id(0)
    t = pl.num_programs(0)

    def copy_in(step, slot):
        return pltpu.make_async_copy(
            adj_hbm.at[pl.ds(step * _BM, _BM), :],
            adj_buf.at[slot],
            sem.at[slot],
        )

    @pl.when(i == 0)
    def _warmup():
        for s in range(_SLOTS):
            copy_in(s, s).start()
        support_ref[...] = jnp.dot(
            x_ref[...], w_ref[...], preferred_element_type=jnp.float32
        )

    slot = jax.lax.rem(i, _SLOTS)
    copy_in(i, slot).wait()
    out_ref[...] = jnp.dot(
        adj_buf[slot], support_ref[...], preferred_element_type=jnp.float32
    )

    @pl.when(i + _SLOTS < t)
    def _prefetch():
        copy_in(i + _SLOTS, slot).start()


def kernel(x, adjacency, W):
    n, d_in = x.shape
    d_out = W.shape[1]
    bm = _BM
    return pl.pallas_call(
        _gcn_manual,
        grid=(n // bm,),
        in_specs=[
            pl.BlockSpec((n, d_in), lambda i: (0, 0)),
            pl.BlockSpec((d_in, d_out), lambda i: (0, 0)),
            pl.BlockSpec(memory_space=pl.MemorySpace.ANY),
        ],
        out_specs=pl.BlockSpec((bm, d_out), lambda i: (i, 0)),
        out_shape=jax.ShapeDtypeStruct((n, d_out), jnp.float32),
        scratch_shapes=[
            pltpu.VMEM((n, d_out), jnp.float32),
            pltpu.VMEM((_SLOTS, bm, n), jnp.float32),
            pltpu.SemaphoreType.DMA((_SLOTS,)),
        ],
        compiler_params=pltpu.CompilerParams(
            dimension_semantics=("arbitrary",),
        ),
    )(x, W, adjacency)


# speedup vs baseline: 1.0159x; 1.0159x over previous
import jax
import jax.numpy as jnp
from jax.experimental import pallas as pl
from jax.experimental.pallas import tpu as pltpu

_BM = 400


def _gcn_fused(x_ref, w_ref, adj_ref, out_ref, support_ref):
    @pl.when(pl.program_id(0) == 0)
    def _compute_support():
        support_ref[...] = jnp.dot(
            x_ref[...], w_ref[...], preferred_element_type=jnp.float32
        )

    out_ref[...] = jnp.dot(
        adj_ref[...], support_ref[...], preferred_element_type=jnp.float32
    )


def kernel(x, adjacency, W):
    n, d_in = x.shape
    d_out = W.shape[1]
    bm = _BM
    return pl.pallas_call(
        _gcn_fused,
        grid=(n // bm,),
        in_specs=[
            pl.BlockSpec((n, d_in), lambda i: (0, 0)),
            pl.BlockSpec((d_in, d_out), lambda i: (0, 0)),
            pl.BlockSpec((bm, n), lambda i: (i, 0)),
        ],
        out_specs=pl.BlockSpec((bm, d_out), lambda i: (i, 0)),
        out_shape=jax.ShapeDtypeStruct((n, d_out), jnp.float32),
        scratch_shapes=[pltpu.VMEM((n, d_out), jnp.float32)],
        compiler_params=pltpu.CompilerParams(
            dimension_semantics=("arbitrary",),
        ),
    )(x, W, adjacency)
